# baseline (speedup 1.0000x reference)
import functools

import jax
import jax.numpy as jnp
import numpy as np
from jax import lax
from jax.experimental import pallas as pl
from jax.experimental.pallas import tpu as pltpu
from jax.experimental.pallas import tpu_sc as plsc

B = 16384
F = 26
NV = 200
K = F * NV
NC = 2
NS = 16
NW = NC * NS
RW = B // NW
CH = 16
NCHUNK = RW // CH
CW = CH * K
XW = RW * F
VPC = CH * F // 16


def _make_sc_onehot():
    mesh = plsc.VectorSubcoreMesh(core_axis_name="c", subcore_axis_name="s")

    @functools.partial(
        pl.kernel,
        mesh=mesh,
        out_type=jax.ShapeDtypeStruct((B * K,), jnp.float32),
        scratch_types=[
            pltpu.VMEM((XW,), jnp.int32),
            pltpu.VMEM((CH * F,), jnp.int32),
            pltpu.VMEM((CW,), jnp.float32),
        ],
        compiler_params=pltpu.CompilerParams(needs_layout_passes=False),
    )
    def onehot(xe_hbm, base_hbm, out_hbm, xe_v, base_v, buf):
        wid = lax.axis_index("s") * NC + lax.axis_index("c")
        pltpu.sync_copy(xe_hbm.at[pl.ds(wid * XW, XW)], xe_v)
        pltpu.sync_copy(base_hbm, base_v)

        zeros16 = jnp.zeros((16,), jnp.float32)
        ones16 = jnp.ones((16,), jnp.float32)

        def zero_body(i, carry):
            buf[pl.ds(i * 16, 16)] = zeros16
            return carry

        lax.fori_loop(0, CW // 16, zero_body, 0)

        def chunk_body(g, carry):
            xoff = g * (CH * F)
            for v in range(VPC):
                val = xe_v[pl.ds(xoff + v * 16, 16)]
                plsc.store_scatter(buf, [base_v[pl.ds(v * 16, 16)] + val],
                                   ones16)
            pltpu.sync_copy(buf, out_hbm.at[pl.ds((wid * RW + g * CH) * K, CW)])
            for v in range(VPC):
                val = xe_v[pl.ds(xoff + v * 16, 16)]
                plsc.store_scatter(buf, [base_v[pl.ds(v * 16, 16)] + val],
                                   zeros16)
            return carry

        lax.fori_loop(0, NCHUNK, chunk_body, 0)

    return onehot


_sc_onehot = _make_sc_onehot()

_BASE = np.asarray(
    [(q // F) * K + (q % F) * NV for q in range(CH * F)], dtype=np.int32)


@jax.jit
def kernel(xe):
    out = _sc_onehot(xe.reshape(B * F), jnp.asarray(_BASE))
    return out.reshape(B, K)


# speedup vs baseline: 1.6020x; 1.6020x over previous
import functools

import jax
import jax.numpy as jnp
import numpy as np
from jax import lax
from jax.experimental import pallas as pl
from jax.experimental.pallas import tpu as pltpu
from jax.experimental.pallas import tpu_sc as plsc

B = 16384
F = 26
NV = 200
K = F * NV
NC = 2
NS = 16
NW = NC * NS
RW = B // NW
CH = 16
NCHUNK = RW // CH
XW = RW * F
VPC = CH * F // 16


def _make_sc_onehot():
    mesh = plsc.VectorSubcoreMesh(core_axis_name="c", subcore_axis_name="s")

    @functools.partial(
        pl.kernel,
        mesh=mesh,
        out_type=jax.ShapeDtypeStruct((B, K), jnp.float32),
        scratch_types=[
            pltpu.VMEM((XW,), jnp.int32),
            pltpu.VMEM((CH * F,), jnp.int32),
            pltpu.VMEM((CH * F,), jnp.int32),
            pltpu.VMEM((CH, K), jnp.float32),
        ],
        compiler_params=pltpu.CompilerParams(needs_layout_passes=False),
    )
    def onehot(xe_hbm, rowt_hbm, colb_hbm, out_hbm, xe_v, rowt_v, colb_v, buf):
        wid = lax.axis_index("s") * NC + lax.axis_index("c")
        pltpu.sync_copy(xe_hbm.at[pl.ds(wid * XW, XW)], xe_v)
        pltpu.sync_copy(rowt_hbm, rowt_v)
        pltpu.sync_copy(colb_hbm, colb_v)

        zeros16 = jnp.zeros((16,), jnp.float32)
        ones16 = jnp.ones((16,), jnp.float32)

        def zero_body(i, carry):
            def zero_row(j, carry2):
                buf[i, pl.ds(j * 16, 16)] = zeros16
                return carry2
            return lax.fori_loop(0, K // 16, zero_row, carry)

        lax.fori_loop(0, CH, zero_body, 0)

        def chunk_body(g, carry):
            xoff = g * (CH * F)
            for v in range(VPC):
                val = xe_v[pl.ds(xoff + v * 16, 16)]
                row = rowt_v[pl.ds(v * 16, 16)]
                col = colb_v[pl.ds(v * 16, 16)] + val
                plsc.store_scatter(buf, [row, col], ones16)
            pltpu.sync_copy(buf, out_hbm.at[pl.ds(wid * RW + g * CH, CH)])
            for v in range(VPC):
                val = xe_v[pl.ds(xoff + v * 16, 16)]
                row = rowt_v[pl.ds(v * 16, 16)]
                col = colb_v[pl.ds(v * 16, 16)] + val
                plsc.store_scatter(buf, [row, col], zeros16)
            return carry

        lax.fori_loop(0, NCHUNK, chunk_body, 0)

    return onehot


_sc_onehot = _make_sc_onehot()

_ROWT = np.asarray([q // F for q in range(CH * F)], dtype=np.int32)
_COLB = np.asarray([(q % F) * NV for q in range(CH * F)], dtype=np.int32)


@jax.jit
def kernel(xe):
    return _sc_onehot(xe.reshape(B * F), jnp.asarray(_ROWT),
                      jnp.asarray(_COLB))


# speedup vs baseline: 4.3830x; 2.7359x over previous
import functools

import jax
import jax.numpy as jnp
from jax import lax
from jax.experimental import pallas as pl
from jax.experimental.pallas import tpu as pltpu
from jax.experimental.pallas import tpu_sc as plsc

B = 16384
F = 26
NV = 200
K = F * NV
NC = 2
NS = 16
NW = NC * NS
CB = B // NW
XW = CB * F
VPB = CB // 16


def _make_sc_onehot():
    mesh = plsc.VectorSubcoreMesh(core_axis_name="c", subcore_axis_name="s")

    @functools.partial(
        pl.kernel,
        mesh=mesh,
        out_type=jax.ShapeDtypeStruct((K, B), jnp.float32),
        scratch_types=[
            pltpu.VMEM((XW,), jnp.int32),
            pltpu.VMEM((NV, CB), jnp.float32),
        ],
        compiler_params=pltpu.CompilerParams(needs_layout_passes=False),
    )
    def onehot(xe_hbm, out_hbm, xe_v, buf):
        wid = lax.axis_index("s") * NC + lax.axis_index("c")
        pltpu.sync_copy(xe_hbm.at[pl.ds(wid * XW, XW)], xe_v)

        zeros16 = jnp.zeros((16,), jnp.float32)
        ones16 = jnp.ones((16,), jnp.float32)
        iota16 = lax.iota(jnp.int32, 16)

        def zero_body(i, carry):
            def zero_row(j, carry2):
                buf[i, pl.ds(j * 16, 16)] = zeros16
                return carry2
            return lax.fori_loop(0, CB // 16, zero_row, carry)

        lax.fori_loop(0, NV, zero_body, 0)

        def field_body(f, carry):
            for k in range(VPB):
                idx = iota16 * F + (k * 16 * F) + f
                val = plsc.load_gather(xe_v, [idx])
                plsc.store_scatter(buf, [val, iota16 + k * 16], ones16)
            pltpu.sync_copy(
                buf, out_hbm.at[pl.ds(f * NV, NV), pl.ds(wid * CB, CB)])
            for k in range(VPB):
                idx = iota16 * F + (k * 16 * F) + f
                val = plsc.load_gather(xe_v, [idx])
                plsc.store_scatter(buf, [val, iota16 + k * 16], zeros16)
            return carry

        lax.fori_loop(0, F, field_body, 0)

    return onehot


_sc_onehot = _make_sc_onehot()


@jax.jit
def kernel(xe):
    return _sc_onehot(xe.reshape(B * F)).T


# speedup vs baseline: 4.8794x; 1.1133x over previous
import functools

import jax
import jax.numpy as jnp
from jax import lax
from jax.experimental import pallas as pl
from jax.experimental.pallas import tpu as pltpu
from jax.experimental.pallas import tpu_sc as plsc

B = 16384
F = 26
NV = 200
K = F * NV
NC = 2
NS = 16
NW = NC * NS
CB = B // NW
VPB = CB // 16
RA = 96
RB = NV - RA


def _make_sc_onehot():
    mesh = plsc.VectorSubcoreMesh(core_axis_name="c", subcore_axis_name="s")

    @functools.partial(
        pl.kernel,
        mesh=mesh,
        out_type=jax.ShapeDtypeStruct((K, B), jnp.float32),
        scratch_types=[
            pltpu.VMEM((CB * F,), jnp.int32),
            pltpu.VMEM((RA, CB), jnp.float32),
            pltpu.VMEM((RB, CB), jnp.float32),
            pltpu.SemaphoreType.DMA,
            pltpu.SemaphoreType.DMA,
        ],
        compiler_params=pltpu.CompilerParams(needs_layout_passes=False),
    )
    def onehot(xe_hbm, out_hbm, xe_v, buf_a, buf_b, sem_a, sem_b):
        wid = lax.axis_index("s") * NC + lax.axis_index("c")
        col0 = wid * CB
        pltpu.sync_copy(xe_hbm.at[pl.ds(col0 * F, CB * F)], xe_v)

        zeros16 = jnp.zeros((16,), jnp.float32)
        ones16 = jnp.ones((16,), jnp.float32)
        iota16 = lax.iota(jnp.int32, 16)

        def zero_a(i, carry):
            for j in range(CB // 16):
                buf_a[i, pl.ds(j * 16, 16)] = zeros16
            return carry

        def zero_b(i, carry):
            for j in range(CB // 16):
                buf_b[i, pl.ds(j * 16, 16)] = zeros16
            return carry

        lax.fori_loop(0, RA, zero_a, 0)
        lax.fori_loop(0, RB, zero_b, 0)

        def scatter_ones(f):
            for k in range(VPB):
                bcol = iota16 + k * 16
                val = plsc.load_gather(xe_v, [bcol * F + f])
                plsc.store_scatter(buf_a, [val, bcol], ones16, mask=val < RA)
                plsc.store_scatter(buf_b, [val - RA, bcol], ones16,
                                   mask=val >= RA)

        def scatter_zeros(f):
            for k in range(VPB):
                bcol = iota16 + k * 16
                val = plsc.load_gather(xe_v, [bcol * F + f])
                plsc.store_scatter(buf_a, [jnp.minimum(val, RA - 1), bcol],
                                   zeros16)
                plsc.store_scatter(buf_b, [jnp.maximum(val, RA) - RA, bcol],
                                   zeros16)

        def fire(f):
            pltpu.async_copy(
                buf_a, out_hbm.at[pl.ds(f * NV, RA), pl.ds(col0, CB)], sem_a)
            pltpu.async_copy(
                buf_b, out_hbm.at[pl.ds(f * NV + RA, RB), pl.ds(col0, CB)],
                sem_b)

        def drain(f):
            pltpu.make_async_copy(
                buf_a, out_hbm.at[pl.ds(f * NV, RA), pl.ds(col0, CB)],
                sem_a).wait()
            pltpu.make_async_copy(
                buf_b, out_hbm.at[pl.ds(f * NV + RA, RB), pl.ds(col0, CB)],
                sem_b).wait()

        scatter_ones(jnp.int32(0))
        fire(jnp.int32(0))

        def field_body(f, carry):
            drain(f - 1)
            scatter_zeros(f - 1)
            scatter_ones(f)
            fire(f)
            return carry

        lax.fori_loop(1, F, field_body, 0)
        drain(jnp.int32(F - 1))

    return onehot


_sc_onehot = _make_sc_onehot()


@jax.jit
def kernel(xe):
    return _sc_onehot(xe.reshape(B * F)).T


# speedup vs baseline: 5.2613x; 1.0783x over previous
import functools

import jax
import jax.numpy as jnp
from jax import lax
from jax.experimental import pallas as pl
from jax.experimental.pallas import tpu as pltpu
from jax.experimental.pallas import tpu_sc as plsc

B = 16384
F = 26
NV = 200
K = F * NV
NC = 2
NS = 16
NW = NC * NS
CB = B // NW
VPB = CB // 16
RA = 96
RB = NV - RA


def _make_sc_onehot():
    mesh = plsc.VectorSubcoreMesh(core_axis_name="c", subcore_axis_name="s")

    @functools.partial(
        pl.kernel,
        mesh=mesh,
        out_type=jax.ShapeDtypeStruct((K, B), jnp.float32),
        scratch_types=[
            pltpu.VMEM((CB * F,), jnp.int32),
            pltpu.VMEM((RA, CB), jnp.float32),
            pltpu.VMEM((RB, CB), jnp.float32),
            pltpu.SemaphoreType.DMA,
            pltpu.SemaphoreType.DMA,
            pltpu.SemaphoreType.DMA,
        ],
        compiler_params=pltpu.CompilerParams(needs_layout_passes=False),
    )
    def onehot(xe_hbm, out_hbm, xe_v, buf_a, buf_b, sem_a, sem_b, sem_x):
        wid = lax.axis_index("s") * NC + lax.axis_index("c")
        col0 = wid * CB
        xe_copy = pltpu.async_copy(
            xe_hbm.at[pl.ds(col0 * F, CB * F)], xe_v, sem_x)

        zeros16 = jnp.zeros((16,), jnp.float32)
        ones16 = jnp.ones((16,), jnp.float32)
        iota16 = lax.iota(jnp.int32, 16)

        def zero_a(i, carry):
            for j in range(CB // 16):
                buf_a[i, pl.ds(j * 16, 16)] = zeros16
            return carry

        def zero_b(i, carry):
            for j in range(CB // 16):
                buf_b[i, pl.ds(j * 16, 16)] = zeros16
            return carry

        def ones_a(f):
            for k in range(VPB):
                bcol = iota16 + k * 16
                val = plsc.load_gather(xe_v, [bcol * F + f])
                plsc.store_scatter(buf_a, [val, bcol], ones16, mask=val < RA)

        def ones_b(f):
            for k in range(VPB):
                bcol = iota16 + k * 16
                val = plsc.load_gather(xe_v, [bcol * F + f])
                plsc.store_scatter(buf_b, [val - RA, bcol], ones16,
                                   mask=val >= RA)

        def clear_a(f):
            for k in range(VPB):
                bcol = iota16 + k * 16
                val = plsc.load_gather(xe_v, [bcol * F + f])
                plsc.store_scatter(buf_a, [jnp.minimum(val, RA - 1), bcol],
                                   zeros16)

        def clear_b(f):
            for k in range(VPB):
                bcol = iota16 + k * 16
                val = plsc.load_gather(xe_v, [bcol * F + f])
                plsc.store_scatter(buf_b, [jnp.maximum(val, RA) - RA, bcol],
                                   zeros16)

        def fire_a(f):
            pltpu.async_copy(
                buf_a, out_hbm.at[pl.ds(f * NV, RA), pl.ds(col0, CB)], sem_a)

        def fire_b(f):
            pltpu.async_copy(
                buf_b, out_hbm.at[pl.ds(f * NV + RA, RB), pl.ds(col0, CB)],
                sem_b)

        def drain_a(f):
            pltpu.make_async_copy(
                buf_a, out_hbm.at[pl.ds(f * NV, RA), pl.ds(col0, CB)],
                sem_a).wait()

        def drain_b(f):
            pltpu.make_async_copy(
                buf_b, out_hbm.at[pl.ds(f * NV + RA, RB), pl.ds(col0, CB)],
                sem_b).wait()

        lax.fori_loop(0, RA, zero_a, 0)
        xe_copy.wait()
        ones_a(jnp.int32(0))
        fire_a(jnp.int32(0))
        lax.fori_loop(0, RB, zero_b, 0)
        ones_b(jnp.int32(0))
        fire_b(jnp.int32(0))

        def field_body(f, carry):
            drain_a(f - 1)
            clear_a(f - 1)
            ones_a(f)
            fire_a(f)
            drain_b(f - 1)
            clear_b(f - 1)
            ones_b(f)
            fire_b(f)
            return carry

        lax.fori_loop(1, F, field_body, 0)
        drain_a(jnp.int32(F - 1))
        drain_b(jnp.int32(F - 1))

    return onehot


_sc_onehot = _make_sc_onehot()


@jax.jit
def kernel(xe):
    return _sc_onehot(xe.reshape(B * F)).T
